# Initial kernel scaffold; baseline (speedup 1.0000x reference)
import functools
import jax
import jax.numpy as jnp
from jax import lax
from jax.experimental import pallas as pl
from jax.experimental.pallas import tpu as pltpu
from jax.experimental.pallas import tpu_sc as plsc

N = 10000
E = 320000
D = 128
H = 64
NC = 2
NS = 16
L = 16
NW = NC * NS

CH = 128
EPT = E // NS
NCHUNK = -(-EPT // CH)
EPT_PAD = NCHUNK * CH
EPW = E // NW
PAD_ROWS = 8
RPT = N // NS

_sc_mesh = plsc.VectorSubcoreMesh(core_axis_name="c", subcore_axis_name="s")


@functools.partial(
    pl.kernel,
    out_type=jax.ShapeDtypeStruct((NW, N), jnp.float32),
    mesh=_sc_mesh,
    scratch_types=[
        pltpu.VMEM((EPW,), jnp.int32),
        pltpu.VMEM((N,), jnp.float32),
    ],
)
def _deg_kernel(dst_hbm, out_hbm, idx_v, hist_v):
    c = lax.axis_index("c")
    s = lax.axis_index("s")
    w = c * NS + s
    pltpu.sync_copy(dst_hbm.at[w], idx_v)

    zeros = jnp.zeros((L,), jnp.float32)
    ones = jnp.ones((L,), jnp.float32)

    def zero_body(i, _):
        hist_v[pl.ds(i * L, L)] = zeros
        return 0

    lax.fori_loop(0, N // L, zero_body, 0)

    def add_body(i, _):
        idx = idx_v[pl.ds(i * L, L)]
        plsc.addupdate_scatter(hist_v, [idx], ones)
        return 0

    lax.fori_loop(0, EPW // L, add_body, 0)
    pltpu.sync_copy(hist_v, out_hbm.at[w])


def _mm_body(x_ref, w_ref, degp_ref, g2_ref, dinv_ref):
    deg = jnp.sum(degp_ref[...], axis=0) + 1.0
    dinv = lax.rsqrt(deg)
    h = jnp.dot(x_ref[...], w_ref[...], preferred_element_type=jnp.float32)
    g = h * dinv[:, None]
    g2_ref[0] = g[:, :H]
    g2_ref[1] = g[:, H:]
    dinv_ref[...] = dinv[:, None]


_R = 1000


def _matmul_scale(x, W, deg_partials):
    return pl.pallas_call(
        _mm_body,
        grid=(N // _R,),
        in_specs=[
            pl.BlockSpec((_R, D), lambda i: (i, 0)),
            pl.BlockSpec((D, D), lambda i: (0, 0)),
            pl.BlockSpec((NW, _R), lambda i: (0, i)),
        ],
        out_specs=[
            pl.BlockSpec((NC, _R, H), lambda i: (0, i, 0)),
            pl.BlockSpec((_R, 1), lambda i: (i, 0)),
        ],
        out_shape=[
            jax.ShapeDtypeStruct((NC, N, H), jnp.float32),
            jax.ShapeDtypeStruct((N, 1), jnp.float32),
        ],
    )(x, W, deg_partials)


@functools.partial(
    pl.kernel,
    out_type=jax.ShapeDtypeStruct((NC, N, H), jnp.float32),
    mesh=_sc_mesh,
    scratch_types=[
        pltpu.VMEM((NCHUNK, CH), jnp.int32),
        pltpu.VMEM((NCHUNK, CH), jnp.int32),
        pltpu.VMEM((CH, H), jnp.float32),
        pltpu.VMEM_SHARED((N + PAD_ROWS, H), jnp.float32),
    ],
)
def _edge_kernel(g2_hbm, src_hbm, dst_hbm, out_hbm, src_v, dst_v, rows_v,
                 acc_sh):
    c = lax.axis_index("c")
    s = lax.axis_index("s")
    g_view = g2_hbm.at[c]

    pltpu.sync_copy(src_hbm.at[s], src_v)
    pltpu.sync_copy(dst_hbm.at[s], dst_v)
    pltpu.sync_copy(g_view.at[pl.ds(s * RPT, RPT)],
                    acc_sh.at[pl.ds(s * RPT, RPT)])
    plsc.subcore_barrier()

    def chunk_body(j, _):
        pltpu.sync_copy(g_view.at[src_v.at[j]], rows_v)
        pltpu.sync_copy(rows_v, acc_sh.at[dst_v.at[j]], add=True)
        return 0

    lax.fori_loop(0, NCHUNK, chunk_body, 0)
    plsc.subcore_barrier()
    pltpu.sync_copy(acc_sh.at[pl.ds(s * RPT, RPT)],
                    out_hbm.at[c].at[pl.ds(s * RPT, RPT)])


def _fin_body(acc_ref, dinv_ref, b_ref, out_ref):
    h = jnp.concatenate([acc_ref[0], acc_ref[1]], axis=-1)
    out_ref[...] = jnp.maximum(h * dinv_ref[...] + b_ref[...], 0.0)


def _finalize(acc2, dinv, b):
    return pl.pallas_call(
        _fin_body,
        grid=(N // _R,),
        in_specs=[
            pl.BlockSpec((NC, _R, H), lambda i: (0, i, 0)),
            pl.BlockSpec((_R, 1), lambda i: (i, 0)),
            pl.BlockSpec((1, D), lambda i: (0, 0)),
        ],
        out_specs=pl.BlockSpec((_R, D), lambda i: (i, 0)),
        out_shape=jax.ShapeDtypeStruct((N, D), jnp.float32),
    )(acc2, dinv, b)


def kernel(x, edge_index, W, b):
    src = edge_index[0]
    dst = edge_index[1]

    dst_a = dst.reshape(NW, EPW)
    deg_partials = _deg_kernel(dst_a)

    g2, dinv = _matmul_scale(x, W, deg_partials)

    pad = EPT_PAD - EPT
    src_c = jnp.pad(src.reshape(NS, EPT), ((0, 0), (0, pad)),
                    constant_values=0).reshape(NS, NCHUNK, CH)
    dst_c = jnp.pad(dst.reshape(NS, EPT), ((0, 0), (0, pad)),
                    constant_values=N).reshape(NS, NCHUNK, CH)

    acc2 = _edge_kernel(g2, src_c, dst_c)
    return _finalize(acc2, dinv, b)


# speedup vs baseline: 23.4954x; 23.4954x over previous
import functools
import jax
import jax.numpy as jnp
from jax import lax
from jax.experimental import pallas as pl
from jax.experimental.pallas import tpu as pltpu
from jax.experimental.pallas import tpu_sc as plsc

N = 10000
E = 320000
D = 128
H = 64
NC = 2
NS = 16
L = 16
NW = NC * NS

CH = 128
EPT = E // NS
NCHUNK = -(-EPT // CH)
EPT_PAD = NCHUNK * CH
EPW = E // NW
PAD_ROWS = 8
RPT = N // NS

_sc_mesh = plsc.VectorSubcoreMesh(core_axis_name="c", subcore_axis_name="s")
_sc_params = pltpu.CompilerParams(needs_layout_passes=False,
                                  use_tc_tiling_on_sc=False)


@functools.partial(
    pl.kernel,
    out_type=jax.ShapeDtypeStruct((NW, N), jnp.float32),
    mesh=_sc_mesh,
    scratch_types=[
        pltpu.VMEM((EPW,), jnp.int32),
        pltpu.VMEM((N,), jnp.float32),
    ],
    compiler_params=_sc_params,
)
def _deg_kernel(dst_hbm, out_hbm, idx_v, hist_v):
    c = lax.axis_index("c")
    s = lax.axis_index("s")
    w = c * NS + s
    pltpu.sync_copy(dst_hbm.at[w], idx_v)

    zeros = jnp.zeros((L,), jnp.float32)
    ones = jnp.ones((L,), jnp.float32)

    def zero_body(i, _):
        hist_v[pl.ds(i * L, L)] = zeros
        return 0

    lax.fori_loop(0, N // L, zero_body, 0)

    def add_body(i, _):
        idx = idx_v[pl.ds(i * L, L)]
        plsc.addupdate_scatter(hist_v, [idx], ones)
        return 0

    lax.fori_loop(0, EPW // L, add_body, 0)
    pltpu.sync_copy(hist_v, out_hbm.at[w])


def _mm_body(x_ref, w_ref, degp_ref, g2_ref, dinv_ref):
    deg = jnp.sum(degp_ref[...], axis=1) + 1.0
    dinv = lax.rsqrt(deg)
    h = jnp.dot(x_ref[...], w_ref[...], preferred_element_type=jnp.float32)
    g = h * dinv[:, None]
    g2_ref[0] = g[:, :H]
    g2_ref[1] = g[:, H:]
    dinv_ref[...] = dinv[:, None]


_R = 1000


def _matmul_scale(x, W, deg_partials):
    return pl.pallas_call(
        _mm_body,
        grid=(N // _R,),
        in_specs=[
            pl.BlockSpec((_R, D), lambda i: (i, 0)),
            pl.BlockSpec((D, D), lambda i: (0, 0)),
            pl.BlockSpec((_R, NW), lambda i: (i, 0)),
        ],
        out_specs=[
            pl.BlockSpec((NC, _R, H), lambda i: (0, i, 0)),
            pl.BlockSpec((_R, 1), lambda i: (i, 0)),
        ],
        out_shape=[
            jax.ShapeDtypeStruct((NC, N, H), jnp.float32),
            jax.ShapeDtypeStruct((N, 1), jnp.float32),
        ],
    )(x, W, deg_partials)


@functools.partial(
    pl.kernel,
    out_type=jax.ShapeDtypeStruct((NC, N, H), jnp.float32),
    mesh=_sc_mesh,
    scratch_types=[
        pltpu.VMEM((NCHUNK, CH), jnp.int32),
        pltpu.VMEM((NCHUNK, CH), jnp.int32),
        pltpu.VMEM((CH, H), jnp.float32),
        pltpu.VMEM_SHARED((N + PAD_ROWS, H), jnp.float32),
    ],
    compiler_params=_sc_params,
)
def _edge_kernel(g2_hbm, src_hbm, dst_hbm, out_hbm, src_v, dst_v, rows_v,
                 acc_sh):
    c = lax.axis_index("c")
    s = lax.axis_index("s")
    g_view = g2_hbm.at[c]

    pltpu.sync_copy(src_hbm.at[s], src_v)
    pltpu.sync_copy(dst_hbm.at[s], dst_v)
    pltpu.sync_copy(g_view.at[pl.ds(s * RPT, RPT)],
                    acc_sh.at[pl.ds(s * RPT, RPT)])
    plsc.subcore_barrier()

    def chunk_body(j, _):
        pltpu.sync_copy(g_view.at[src_v.at[j]], rows_v)
        pltpu.sync_copy(rows_v, acc_sh.at[dst_v.at[j]], add=True)
        return 0

    lax.fori_loop(0, NCHUNK, chunk_body, 0)
    plsc.subcore_barrier()
    pltpu.sync_copy(acc_sh.at[pl.ds(s * RPT, RPT)],
                    out_hbm.at[c].at[pl.ds(s * RPT, RPT)])


def _fin_body(acc_ref, dinv_ref, b_ref, out_ref):
    h = jnp.concatenate([acc_ref[0], acc_ref[1]], axis=-1)
    out_ref[...] = jnp.maximum(h * dinv_ref[...] + b_ref[...], 0.0)


def _finalize(acc2, dinv, b):
    return pl.pallas_call(
        _fin_body,
        grid=(N // _R,),
        in_specs=[
            pl.BlockSpec((NC, _R, H), lambda i: (0, i, 0)),
            pl.BlockSpec((_R, 1), lambda i: (i, 0)),
            pl.BlockSpec((1, D), lambda i: (0, 0)),
        ],
        out_specs=pl.BlockSpec((_R, D), lambda i: (i, 0)),
        out_shape=jax.ShapeDtypeStruct((N, D), jnp.float32),
    )(acc2, dinv, b.reshape(1, D))


def kernel(x, edge_index, W, b):
    src = edge_index[0]
    dst = edge_index[1]

    dst_a = dst.reshape(NW, EPW)
    deg_partials = _deg_kernel(dst_a)

    g2, dinv = _matmul_scale(x, W, deg_partials.T)

    pad = EPT_PAD - EPT
    src_c = jnp.pad(src.reshape(NS, EPT), ((0, 0), (0, pad)),
                    constant_values=0).reshape(NS, NCHUNK, CH)
    dst_c = jnp.pad(dst.reshape(NS, EPT), ((0, 0), (0, pad)),
                    constant_values=N).reshape(NS, NCHUNK, CH)

    acc2 = _edge_kernel(g2, src_c, dst_c)
    return _finalize(acc2, dinv, b)
